# Initial kernel scaffold; baseline (speedup 1.0000x reference)
import jax
import jax.numpy as jnp
from jax.experimental import pallas as pl


def kernel(input, target):
    raise NotImplementedError("write your pallas kernel here")


# speedup vs baseline: 1.2436x; 1.2436x over previous
import functools
import math

import jax
import jax.numpy as jnp
from jax import lax
from jax.experimental import pallas as pl
from jax.experimental.pallas import tpu as pltpu
from jax.experimental.pallas import tpu_sc as plsc

B = 1024
V = 100000
S = 16.0
M_MARGIN = 0.1
COS_M = math.cos(M_MARGIN)
SIN_M = math.sin(M_MARGIN)
COS_PI_M = math.cos(math.pi - M_MARGIN)
SIN_PI_M = math.sin(math.pi - M_MARGIN)

NC = 2
NS = 16
L = 16
NW = NC * NS
BPW = B // NW

CHUNK = 2048
GRID = (V + CHUNK - 1) // CHUNK


def _gather_body(flat_hbm, tgt_hbm, out_hbm, idx_v, val_v, sem):
    wid = lax.axis_index("s") * NC + lax.axis_index("c")
    base = wid * BPW
    pltpu.sync_copy(tgt_hbm.at[pl.ds(base, BPW)], idx_v)
    for i in range(BPW // L):
        row = base + i * L + lax.iota(jnp.int32, L)
        idx_v[pl.ds(i * L, L)] = idx_v[pl.ds(i * L, L)] + row * V
    pltpu.async_copy(flat_hbm.at[idx_v], val_v, sem).wait()
    pltpu.sync_copy(val_v, out_hbm.at[pl.ds(base, BPW)])


def _sc_gather(flat_input, target):
    mesh = plsc.VectorSubcoreMesh(core_axis_name="c", subcore_axis_name="s")
    return pl.kernel(
        _gather_body,
        mesh=mesh,
        out_type=jax.ShapeDtypeStruct((B,), jnp.float32),
        scratch_types=[
            pltpu.VMEM((BPW,), jnp.int32),
            pltpu.VMEM((BPW,), jnp.float32),
            pltpu.SemaphoreType.DMA,
        ],
    )(flat_input, target)


def _lse_body(in_ref, m_out, s_out, m_scr, s_scr):
    i = pl.program_id(0)

    @pl.when(i == 0)
    def _init():
        m_scr[...] = jnp.full_like(m_scr, -jnp.inf)
        s_scr[...] = jnp.zeros_like(s_scr)

    x = in_ref[...] * S
    col = i * CHUNK + lax.broadcasted_iota(jnp.int32, (B, CHUNK), 1)
    x = jnp.where(col < V, x, -jnp.inf)
    bm = jnp.max(x, axis=1, keepdims=True)
    m_old = m_scr[...]
    m_new = jnp.maximum(m_old, bm)
    s_scr[...] = s_scr[...] * jnp.exp(m_old - m_new) + jnp.sum(
        jnp.exp(x - m_new), axis=1, keepdims=True
    )
    m_scr[...] = m_new

    @pl.when(i == GRID - 1)
    def _emit():
        m_out[...] = m_scr[...]
        s_out[...] = s_scr[...]


def _tc_lse(input):
    return pl.pallas_call(
        _lse_body,
        grid=(GRID,),
        in_specs=[pl.BlockSpec((B, CHUNK), lambda i: (0, i))],
        out_specs=[
            pl.BlockSpec((B, 1), lambda i: (0, 0)),
            pl.BlockSpec((B, 1), lambda i: (0, 0)),
        ],
        out_shape=[
            jax.ShapeDtypeStruct((B, 1), jnp.float32),
            jax.ShapeDtypeStruct((B, 1), jnp.float32),
        ],
        scratch_shapes=[
            pltpu.VMEM((B, 1), jnp.float32),
            pltpu.VMEM((B, 1), jnp.float32),
        ],
    )(input)


def _combine_body(m_ref, s_ref, ct_ref, out_ref):
    m = m_ref[...]
    ssum = s_ref[...]
    ct = ct_ref[...]
    sin_t = jnp.sqrt(1.0 - ct * ct)
    phi = ct * COS_M - sin_t * SIN_M
    keep = ct - SIN_PI_M * M_MARGIN
    new_val = jnp.where(ct - COS_PI_M > 0, phi, keep)
    s_adj = ssum - jnp.exp(S * ct - m) + jnp.exp(S * new_val - m)
    logz = m + jnp.log(s_adj)
    nll = logz - S * new_val
    out_ref[...] = jnp.sum(nll, keepdims=True).reshape(1, 1) / B


def _tc_combine(m, ssum, cos_t):
    return pl.pallas_call(
        _combine_body,
        out_shape=jax.ShapeDtypeStruct((1, 1), jnp.float32),
    )(m, ssum, cos_t)


@jax.jit
def kernel(input, target):
    target = target.astype(jnp.int32)
    flat = input.reshape(B * V)
    cos_t = _sc_gather(flat, target)
    m, ssum = _tc_lse(input)
    loss = _tc_combine(m, ssum, cos_t.reshape(B, 1))
    return loss[0, 0]


# speedup vs baseline: 2.6319x; 2.1164x over previous
import math

import jax
import jax.numpy as jnp
from jax import lax
from jax.experimental import pallas as pl
from jax.experimental.pallas import tpu as pltpu
from jax.experimental.pallas import tpu_sc as plsc

B = 1024
V = 100000
S = 16.0
SHIFT = 16.0
M_MARGIN = 0.1
COS_M = math.cos(M_MARGIN)
SIN_M = math.sin(M_MARGIN)
COS_PI_M = math.cos(math.pi - M_MARGIN)
SIN_PI_M = math.sin(math.pi - M_MARGIN)

NC = 2
NS = 16
L = 16
NW = NC * NS
BPW = B // NW
W = 16

CHUNK = 2048
GRID = (V + CHUNK - 1) // CHUNK


TAIL0 = (V // 128) * 128
CB_MAX = TAIL0 - 128


def _gather_body(in_hbm, tail_hbm, tgt_hbm, out_hbm, idx_v, win_v, tail_v, val_v, sem):
    wid = lax.axis_index("s") * NC + lax.axis_index("c")
    base = wid * BPW
    pltpu.sync_copy(tgt_hbm.at[pl.ds(base, BPW)], idx_v)
    copies = []
    for rg in range(BPW // 8):
        r0 = pl.multiple_of(base + rg * 8, 8)
        copies.append(
            pltpu.async_copy(
                tail_hbm.at[pl.ds(r0, 8), :], tail_v.at[pl.ds(rg * 8, 8), :], sem
            )
        )
    for g in range(BPW // L):
        cvec = idx_v[pl.ds(g * L, L)]
        cbvec = jnp.minimum((cvec // 128) * 128, CB_MAX)
        for j in range(L):
            i = g * L + j
            r0 = pl.multiple_of(base + (i // 8) * 8, 8)
            cb = pl.multiple_of(cbvec[j], 128)
            copies.append(
                pltpu.async_copy(
                    in_hbm.at[pl.ds(r0, 8), pl.ds(cb, 128)], win_v.at[i], sem
                )
            )
    for cp in copies:
        cp.wait()
    for g in range(BPW // L):
        i16 = g * L + lax.iota(jnp.int32, L)
        cvec = idx_v[pl.ds(g * L, L)]
        cbvec = jnp.minimum((cvec // 128) * 128, CB_MAX)
        rin = lax.rem(i16, 8)
        main_off = jnp.minimum(jnp.maximum(cvec - cbvec, 0), 127)
        v_main = plsc.load_gather(win_v, [i16, rin, main_off])
        tail_off = jnp.minimum(jnp.maximum(cvec - TAIL0, 0), 127)
        v_tail = plsc.load_gather(tail_v, [i16, tail_off])
        val_v[pl.ds(g * L, L)] = jnp.where(cvec >= TAIL0, v_tail, v_main)
    pltpu.sync_copy(val_v, out_hbm.at[pl.ds(base, BPW)])


def _sc_gather(input, tail, target):
    mesh = plsc.VectorSubcoreMesh(core_axis_name="c", subcore_axis_name="s")
    return pl.kernel(
        _gather_body,
        mesh=mesh,
        compiler_params=pltpu.CompilerParams(needs_layout_passes=False),
        out_type=jax.ShapeDtypeStruct((B,), jnp.float32),
        scratch_types=[
            pltpu.VMEM((BPW,), jnp.int32),
            pltpu.VMEM((BPW, 8, 128), jnp.float32),
            pltpu.VMEM((BPW, 128), jnp.float32),
            pltpu.VMEM((BPW,), jnp.float32),
            pltpu.SemaphoreType.DMA,
        ],
    )(input, tail, target)


def _lse_body(in_ref, s_out, s_scr):
    i = pl.program_id(0)

    @pl.when(i == 0)
    def _init():
        s_scr[...] = jnp.zeros_like(s_scr)

    x = in_ref[...]

    @pl.when(i < GRID - 1)
    def _full():
        s_scr[...] += jnp.sum(jnp.exp(x * S - SHIFT), axis=1, keepdims=True)

    @pl.when(i == GRID - 1)
    def _tail():
        col = lax.broadcasted_iota(jnp.int32, (B, CHUNK), 1)
        e = jnp.where(col < V - (GRID - 1) * CHUNK, jnp.exp(x * S - SHIFT), 0.0)
        s_scr[...] += jnp.sum(e, axis=1, keepdims=True)
        s_out[...] = s_scr[...]


def _tc_lse(input):
    return pl.pallas_call(
        _lse_body,
        grid=(GRID,),
        in_specs=[pl.BlockSpec((B, CHUNK), lambda i: (0, i))],
        out_specs=pl.BlockSpec((B, 1), lambda i: (0, 0)),
        out_shape=jax.ShapeDtypeStruct((B, 1), jnp.float32),
        scratch_shapes=[pltpu.VMEM((B, 1), jnp.float32)],
    )(input)


def _combine_body(s_ref, ct_ref, out_ref):
    ssum = s_ref[...]
    ct = ct_ref[...]
    sin_t = jnp.sqrt(1.0 - ct * ct)
    phi = ct * COS_M - sin_t * SIN_M
    keep = ct - SIN_PI_M * M_MARGIN
    new_val = jnp.where(ct - COS_PI_M > 0, phi, keep)
    s_adj = ssum - jnp.exp(S * ct - SHIFT) + jnp.exp(S * new_val - SHIFT)
    logz = SHIFT + jnp.log(s_adj)
    nll = logz - S * new_val
    out_ref[...] = jnp.sum(nll, keepdims=True).reshape(1, 1) / B


def _tc_combine(ssum, cos_t):
    return pl.pallas_call(
        _combine_body,
        out_shape=jax.ShapeDtypeStruct((1, 1), jnp.float32),
    )(ssum, cos_t)


@jax.jit
def kernel(input, target):
    target = target.astype(jnp.int32)
    tail = jnp.pad(input[:, TAIL0:], ((0, 0), (0, 128 - (V - TAIL0))))
    cos_t = _sc_gather(input, tail, target)
    ssum = _tc_lse(input)
    loss = _tc_combine(ssum, cos_t.reshape(B, 1))
    return loss[0, 0]


# speedup vs baseline: 2.6943x; 1.0237x over previous
import math

import jax
import jax.numpy as jnp
from jax import lax
from jax.experimental import pallas as pl
from jax.experimental.pallas import tpu as pltpu
from jax.experimental.pallas import tpu_sc as plsc

B = 1024
V = 100000
S = 16.0
SHIFT = 16.0
M_MARGIN = 0.1
COS_M = math.cos(M_MARGIN)
SIN_M = math.sin(M_MARGIN)
COS_PI_M = math.cos(math.pi - M_MARGIN)
SIN_PI_M = math.sin(math.pi - M_MARGIN)

NC = 2
NS = 16
L = 16
NW = NC * NS
BPW = B // NW
W = 16

CHUNK = 4096
GRID = (V + CHUNK - 1) // CHUNK


TAIL0 = (V // 128) * 128
CB_MAX = TAIL0 - 128


def _gather_body(in_hbm, tail_hbm, tgt_hbm, out_hbm, idx_v, win_v, tail_v, val_v, sem):
    wid = lax.axis_index("s") * NC + lax.axis_index("c")
    base = wid * BPW
    pltpu.sync_copy(tgt_hbm.at[pl.ds(base, BPW)], idx_v)
    copies = []
    for rg in range(BPW // 8):
        r0 = pl.multiple_of(base + rg * 8, 8)
        copies.append(
            pltpu.async_copy(
                tail_hbm.at[pl.ds(r0, 8), :], tail_v.at[pl.ds(rg * 8, 8), :], sem
            )
        )
    for g in range(BPW // L):
        cvec = idx_v[pl.ds(g * L, L)]
        cbvec = jnp.minimum((cvec // 128) * 128, CB_MAX)
        for j in range(L):
            i = g * L + j
            r0 = pl.multiple_of(base + (i // 8) * 8, 8)
            cb = pl.multiple_of(cbvec[j], 128)
            copies.append(
                pltpu.async_copy(
                    in_hbm.at[pl.ds(r0, 8), pl.ds(cb, 128)], win_v.at[i], sem
                )
            )
    for cp in copies:
        cp.wait()
    for g in range(BPW // L):
        i16 = g * L + lax.iota(jnp.int32, L)
        cvec = idx_v[pl.ds(g * L, L)]
        cbvec = jnp.minimum((cvec // 128) * 128, CB_MAX)
        rin = lax.rem(i16, 8)
        main_off = jnp.minimum(jnp.maximum(cvec - cbvec, 0), 127)
        v_main = plsc.load_gather(win_v, [i16, rin, main_off])
        tail_off = jnp.minimum(jnp.maximum(cvec - TAIL0, 0), 127)
        v_tail = plsc.load_gather(tail_v, [i16, tail_off])
        val_v[pl.ds(g * L, L)] = jnp.where(cvec >= TAIL0, v_tail, v_main)
    pltpu.sync_copy(val_v, out_hbm.at[pl.ds(base, BPW)])


def _sc_gather(input, tail, target):
    mesh = plsc.VectorSubcoreMesh(core_axis_name="c", subcore_axis_name="s")
    return pl.kernel(
        _gather_body,
        mesh=mesh,
        compiler_params=pltpu.CompilerParams(needs_layout_passes=False),
        out_type=jax.ShapeDtypeStruct((B,), jnp.float32),
        scratch_types=[
            pltpu.VMEM((BPW,), jnp.int32),
            pltpu.VMEM((BPW, 8, 128), jnp.float32),
            pltpu.VMEM((BPW, 128), jnp.float32),
            pltpu.VMEM((BPW,), jnp.float32),
            pltpu.SemaphoreType.DMA,
        ],
    )(input, tail, target)


def _lse_body(in_ref, s_out, s_scr):
    i = pl.program_id(0)

    @pl.when(i == 0)
    def _init():
        s_scr[...] = jnp.zeros_like(s_scr)

    x = in_ref[...]

    @pl.when(i < GRID - 1)
    def _full():
        s_scr[...] += jnp.sum(jnp.exp(x * S - SHIFT), axis=1, keepdims=True)

    @pl.when(i == GRID - 1)
    def _tail():
        col = lax.broadcasted_iota(jnp.int32, (B, CHUNK), 1)
        e = jnp.where(col < V - (GRID - 1) * CHUNK, jnp.exp(x * S - SHIFT), 0.0)
        s_scr[...] += jnp.sum(e, axis=1, keepdims=True)
        s_out[...] = s_scr[...]


def _tc_lse(input):
    return pl.pallas_call(
        _lse_body,
        grid=(GRID,),
        in_specs=[pl.BlockSpec((B, CHUNK), lambda i: (0, i))],
        out_specs=pl.BlockSpec((B, 1), lambda i: (0, 0)),
        out_shape=jax.ShapeDtypeStruct((B, 1), jnp.float32),
        scratch_shapes=[pltpu.VMEM((B, 1), jnp.float32)],
    )(input)


def _combine_body(s_ref, ct_ref, out_ref):
    ssum = s_ref[...]
    ct = ct_ref[...]
    sin_t = jnp.sqrt(1.0 - ct * ct)
    phi = ct * COS_M - sin_t * SIN_M
    keep = ct - SIN_PI_M * M_MARGIN
    new_val = jnp.where(ct - COS_PI_M > 0, phi, keep)
    s_adj = ssum - jnp.exp(S * ct - SHIFT) + jnp.exp(S * new_val - SHIFT)
    logz = SHIFT + jnp.log(s_adj)
    nll = logz - S * new_val
    out_ref[...] = jnp.sum(nll, keepdims=True).reshape(1, 1) / B


def _tc_combine(ssum, cos_t):
    return pl.pallas_call(
        _combine_body,
        out_shape=jax.ShapeDtypeStruct((1, 1), jnp.float32),
    )(ssum, cos_t)


@jax.jit
def kernel(input, target):
    target = target.astype(jnp.int32)
    tail = jnp.pad(input[:, TAIL0:], ((0, 0), (0, 128 - (V - TAIL0))))
    cos_t = _sc_gather(input, tail, target)
    ssum = _tc_lse(input)
    loss = _tc_combine(ssum, cos_t.reshape(B, 1))
    return loss[0, 0]
